# baseline (speedup 1.0000x reference)
import functools

import jax
import jax.numpy as jnp
from jax import lax
from jax.experimental import pallas as pl
from jax.experimental.pallas import tpu as pltpu
from jax.experimental.pallas import tpu_sc as plsc

N_NODES = 100000
D = 128
B = 4096
N_NEG = 20
ROWS_PER_EDGE = 2 + N_NEG
TOTAL_ROWS = B * ROWS_PER_EDGE

NC = 2
NS = 16
NW = NC * NS

CHUNK = 128
ROWS_PER_TILE = TOTAL_ROWS // NW
CHUNKS_PER_TILE = ROWS_PER_TILE // CHUNK

EDGE_BLK = 128


def _sc_gather(table, idx):
    mesh = plsc.VectorSubcoreMesh(core_axis_name="c", subcore_axis_name="s")

    @functools.partial(
        pl.kernel,
        out_type=jax.ShapeDtypeStruct((TOTAL_ROWS, D), jnp.float32),
        mesh=mesh,
        scratch_types=[
            pltpu.VMEM((CHUNK,), jnp.int32),
            pltpu.VMEM((CHUNK, D), jnp.float32),
            pltpu.SemaphoreType.DMA,
        ],
    )
    def gather_kernel(table_hbm, idx_hbm, out_hbm, idx_v, rows_v, sem):
        wid = lax.axis_index("s") * NC + lax.axis_index("c")
        tile_base = wid * ROWS_PER_TILE

        @pl.loop(0, CHUNKS_PER_TILE)
        def _(c):
            base = tile_base + c * CHUNK
            pltpu.sync_copy(idx_hbm.at[pl.ds(base, CHUNK)], idx_v)
            pltpu.async_copy(table_hbm.at[idx_v], rows_v, sem).wait()
            pltpu.sync_copy(rows_v, out_hbm.at[pl.ds(base, CHUNK)])

    return gather_kernel(table, idx)


def _tc_loss_body(g_ref, w_ref, o_ref):
    i = pl.program_id(0)
    blk = g_ref[...]
    xv = blk[:, 0, :]
    yv = blk[:, 1, :]
    ynv = blk[:, 2:, :]

    dpos = jnp.sum((xv - yv) ** 2, axis=-1)
    dneg = jnp.sum((xv[:, None, :] - ynv) ** 2, axis=-1)

    p_pos = jnp.clip(1.0 / (1.0 + 0.25 * dpos), 1e-12, 1.0 - 1e-12)
    p_neg = jnp.clip(1.0 / (1.0 + 0.25 * dneg), 1e-12, 0.99)
    t_pos = jnp.log(p_pos) * 20.0
    t_neg = jnp.log(1.0 - p_neg)

    loss_b = 7.0 * jnp.sum(t_neg, axis=1) + t_pos
    part = jnp.sum(w_ref[0, :] * loss_b)

    @pl.when(i == 0)
    def _():
        o_ref[...] = jnp.zeros_like(o_ref)

    o_ref[...] += jnp.full((1, 1), -part, jnp.float32)


def _tc_loss(gathered, weights):
    grid = B // EDGE_BLK
    out = pl.pallas_call(
        _tc_loss_body,
        grid=(grid,),
        in_specs=[
            pl.BlockSpec((EDGE_BLK, ROWS_PER_EDGE, D), lambda i: (i, 0, 0)),
            pl.BlockSpec((1, EDGE_BLK), lambda i: (0, i)),
        ],
        out_specs=pl.BlockSpec((1, 1), lambda i: (0, 0)),
        out_shape=jax.ShapeDtypeStruct((1, 1), jnp.float32),
    )(gathered, weights)
    return out[0, 0]


@jax.jit
def kernel(logits, xs, ys, y_neg, sample_edge_weight):
    idx = jnp.concatenate(
        [xs[:, None], ys[:, None], y_neg], axis=1
    ).reshape(-1)
    gathered = _sc_gather(logits, idx)
    gathered = gathered.reshape(B, ROWS_PER_EDGE, D)
    return _tc_loss(gathered, sample_edge_weight.reshape(1, B))


# speedup vs baseline: 2.1528x; 2.1528x over previous
import functools

import jax
import jax.numpy as jnp
from jax import lax
from jax.experimental import pallas as pl
from jax.experimental.pallas import tpu as pltpu
from jax.experimental.pallas import tpu_sc as plsc

N_NODES = 100000
D = 128
B = 4096
N_NEG = 20
N_PAIR = N_NEG + 1
ROWS_PER_EDGE = 2 + N_NEG
TOTAL_ROWS = B * ROWS_PER_EDGE

NC = 2
NS = 16
NW = NC * NS
L = 16

EDGES_PER_TILE = B // NW
EDGES_PER_CHUNK = 4
CHUNK_ROWS = EDGES_PER_CHUNK * ROWS_PER_EDGE
CHUNKS_PER_TILE = EDGES_PER_TILE // EDGES_PER_CHUNK
IDX_PER_TILE = EDGES_PER_TILE * ROWS_PER_EDGE

RES_LANES = 512
RES_ROWS = 2 * EDGES_PER_CHUNK

EDGE_BLK = 256


def _sc_pair_partials(table, idx):
    mesh = plsc.VectorSubcoreMesh(core_axis_name="c", subcore_axis_name="s")

    @functools.partial(
        pl.kernel,
        out_type=jax.ShapeDtypeStruct((B, RES_LANES), jnp.float32),
        mesh=mesh,
        scratch_types=[
            pltpu.VMEM((IDX_PER_TILE,), jnp.int32),
            pltpu.VMEM((CHUNK_ROWS, D), jnp.float32),
            pltpu.VMEM((CHUNK_ROWS, D), jnp.float32),
            pltpu.VMEM((RES_ROWS, RES_LANES), jnp.float32),
            pltpu.SemaphoreType.DMA,
            pltpu.SemaphoreType.DMA,
            pltpu.SemaphoreType.DMA,
        ],
    )
    def sc_kernel(table_hbm, idx_hbm, res_hbm, idx_v, rows0, rows1,
                  res_v, sem_g0, sem_g1, sem_w):
        wid = lax.axis_index("s") * NC + lax.axis_index("c")
        edge_base = wid * EDGES_PER_TILE

        pltpu.sync_copy(idx_hbm.at[pl.ds(wid * IDX_PER_TILE, IDX_PER_TILE)],
                        idx_v)

        zeros = jnp.zeros((L,), jnp.float32)
        for r in range(RES_ROWS):
            for c in range(RES_LANES // L):
                res_v[r, pl.ds(c * L, L)] = zeros

        def start_gather(chunk, rows_v, sem):
            return pltpu.async_copy(
                table_hbm.at[idx_v.at[pl.ds(chunk * CHUNK_ROWS, CHUNK_ROWS)]],
                rows_v, sem)

        def compute_chunk(rows_v, res_row_base):
            for e in range(EDGES_PER_CHUNK):
                erow = e * ROWS_PER_EDGE
                xq = [rows_v[erow, pl.ds(k * L, L)] for k in range(D // L)]
                for p in range(N_PAIR):
                    prow = erow + (2 + p if p < N_NEG else 1)
                    acc = None
                    for k in range(D // L):
                        dlt = xq[k] - rows_v[prow, pl.ds(k * L, L)]
                        sq = dlt * dlt
                        acc = sq if acc is None else acc + sq
                    res_v[res_row_base + e, pl.ds(p * L, L)] = acc

        start_gather(0, rows0, sem_g0)
        start_gather(1, rows1, sem_g1)

        @pl.loop(0, CHUNKS_PER_TILE // 2)
        def _(t):
            c0 = 2 * t

            @pl.when(t > 0)
            def _():
                pltpu.make_async_copy(
                    res_v, res_hbm.at[pl.ds(edge_base, RES_ROWS)], sem_w
                ).wait()

            pltpu.make_async_copy(
                table_hbm.at[idx_v.at[pl.ds(c0 * CHUNK_ROWS, CHUNK_ROWS)]],
                rows0, sem_g0).wait()
            compute_chunk(rows0, 0)

            @pl.when(t < CHUNKS_PER_TILE // 2 - 1)
            def _():
                start_gather(c0 + 2, rows0, sem_g0)

            pltpu.make_async_copy(
                table_hbm.at[idx_v.at[pl.ds((c0 + 1) * CHUNK_ROWS,
                                            CHUNK_ROWS)]],
                rows1, sem_g1).wait()
            compute_chunk(rows1, EDGES_PER_CHUNK)

            @pl.when(t < CHUNKS_PER_TILE // 2 - 1)
            def _():
                start_gather(c0 + 3, rows1, sem_g1)

            pltpu.async_copy(
                res_v,
                res_hbm.at[pl.ds(edge_base + t * RES_ROWS, RES_ROWS)],
                sem_w)

        pltpu.make_async_copy(
            res_v, res_hbm.at[pl.ds(edge_base, RES_ROWS)], sem_w).wait()

    return sc_kernel(table, idx)


def _tc_loss_body(r_ref, w_ref, o_ref):
    i = pl.program_id(0)
    blk = r_ref[...]

    row_ids = lax.broadcasted_iota(jnp.int32, (RES_LANES, RES_LANES // L), 0)
    col_ids = lax.broadcasted_iota(jnp.int32, (RES_LANES, RES_LANES // L), 1)
    sel = (row_ids // L == col_ids).astype(jnp.float32)

    dmat = jax.lax.dot_general(
        blk, sel, (((1,), (0,)), ((), ())),
        preferred_element_type=jnp.float32)

    dneg = dmat[:, :N_NEG]
    dpos = dmat[:, N_NEG:N_NEG + 1]

    p_pos = jnp.clip(1.0 / (1.0 + 0.25 * dpos), 1e-12, 1.0 - 1e-12)
    p_neg = jnp.clip(1.0 / (1.0 + 0.25 * dneg), 1e-12, 0.99)
    t_pos = jnp.log(p_pos) * 20.0
    t_neg = jnp.log(1.0 - p_neg)

    loss_b = 7.0 * jnp.sum(t_neg, axis=1) + t_pos[:, 0]
    part = jnp.sum(w_ref[0, :] * loss_b)

    @pl.when(i == 0)
    def _():
        o_ref[...] = jnp.zeros_like(o_ref)

    o_ref[...] += jnp.full((1, 1), -part, jnp.float32)


def _tc_loss(pair_partials, weights):
    grid = B // EDGE_BLK
    out = pl.pallas_call(
        _tc_loss_body,
        grid=(grid,),
        in_specs=[
            pl.BlockSpec((EDGE_BLK, RES_LANES), lambda i: (i, 0)),
            pl.BlockSpec((1, EDGE_BLK), lambda i: (0, i)),
        ],
        out_specs=pl.BlockSpec((1, 1), lambda i: (0, 0)),
        out_shape=jax.ShapeDtypeStruct((1, 1), jnp.float32),
    )(pair_partials, weights)
    return out[0, 0]


@jax.jit
def kernel(logits, xs, ys, y_neg, sample_edge_weight):
    idx = jnp.concatenate(
        [xs[:, None], ys[:, None], y_neg], axis=1
    ).reshape(-1)
    partials = _sc_pair_partials(logits, idx)
    return _tc_loss(partials, sample_edge_weight.reshape(1, B))


# speedup vs baseline: 2.1746x; 1.0102x over previous
import functools

import jax
import jax.numpy as jnp
from jax import lax
from jax.experimental import pallas as pl
from jax.experimental.pallas import tpu as pltpu
from jax.experimental.pallas import tpu_sc as plsc

N_NODES = 100000
D = 128
B = 4096
N_NEG = 20
N_PAIR = N_NEG + 1

NC = 2
NS = 16
NW = NC * NS
L = 16

EDGES_PER_TILE = B // NW
EDGES_PER_CHUNK = 4
CHUNK_NEG_ROWS = EDGES_PER_CHUNK * N_NEG
CHUNKS_PER_TILE = EDGES_PER_TILE // EDGES_PER_CHUNK
NEG_PER_TILE = EDGES_PER_TILE * N_NEG

RES_LANES = 512
RES_ROWS = 2 * EDGES_PER_CHUNK

EDGE_BLK = 512


def _sc_pair_partials(table, xs, ys, y_neg_flat):
    mesh = plsc.VectorSubcoreMesh(core_axis_name="c", subcore_axis_name="s")

    @functools.partial(
        pl.kernel,
        out_type=jax.ShapeDtypeStruct((B, RES_LANES), jnp.float32),
        mesh=mesh,
        scratch_types=[
            pltpu.VMEM((EDGES_PER_TILE,), jnp.int32),
            pltpu.VMEM((EDGES_PER_TILE,), jnp.int32),
            pltpu.VMEM((NEG_PER_TILE,), jnp.int32),
            pltpu.VMEM((2 * EDGES_PER_TILE, D), jnp.float32),
            pltpu.VMEM((CHUNK_NEG_ROWS, D), jnp.float32),
            pltpu.VMEM((CHUNK_NEG_ROWS, D), jnp.float32),
            pltpu.VMEM((RES_ROWS, RES_LANES), jnp.float32),
            pltpu.SemaphoreType.DMA,
            pltpu.SemaphoreType.DMA,
            pltpu.SemaphoreType.DMA,
        ],
    )
    def sc_kernel(table_hbm, xs_hbm, ys_hbm, yneg_hbm, res_hbm,
                  xi_v, yi_v, ni_v, xy_v, neg0, neg1, res_v,
                  sem_xy, sem_g, sem_w):
        wid = lax.axis_index("s") * NC + lax.axis_index("c")
        edge_base = wid * EDGES_PER_TILE

        pltpu.sync_copy(xs_hbm.at[pl.ds(edge_base, EDGES_PER_TILE)], xi_v)
        pltpu.sync_copy(ys_hbm.at[pl.ds(edge_base, EDGES_PER_TILE)], yi_v)
        pltpu.sync_copy(yneg_hbm.at[pl.ds(wid * NEG_PER_TILE, NEG_PER_TILE)],
                        ni_v)

        pltpu.async_copy(table_hbm.at[xi_v],
                         xy_v.at[pl.ds(0, EDGES_PER_TILE)], sem_xy)
        pltpu.async_copy(table_hbm.at[yi_v],
                         xy_v.at[pl.ds(EDGES_PER_TILE, EDGES_PER_TILE)],
                         sem_xy)

        zeros = jnp.zeros((L,), jnp.float32)
        for r in range(RES_ROWS):
            for c in range(RES_LANES // L):
                res_v[r, pl.ds(c * L, L)] = zeros

        def start_neg_gather(chunk, buf):
            return pltpu.async_copy(
                table_hbm.at[ni_v.at[pl.ds(chunk * CHUNK_NEG_ROWS,
                                           CHUNK_NEG_ROWS)]],
                buf, sem_g)

        start_neg_gather(0, neg0)
        start_neg_gather(1, neg1)
        pltpu.make_async_copy(table_hbm.at[xi_v],
                              xy_v.at[pl.ds(0, EDGES_PER_TILE)],
                              sem_xy).wait()
        pltpu.make_async_copy(table_hbm.at[xi_v],
                              xy_v.at[pl.ds(0, EDGES_PER_TILE)],
                              sem_xy).wait()

        def compute_chunk(t, chunk_parity, neg_v, res_row_base):
            for e in range(EDGES_PER_CHUNK):
                edge = (2 * t + chunk_parity) * EDGES_PER_CHUNK + e
                xq = [xy_v[edge, pl.ds(k * L, L)] for k in range(D // L)]
                for p in range(N_NEG):
                    prow = e * N_NEG + p
                    acc = None
                    for k in range(D // L):
                        dlt = xq[k] - neg_v[prow, pl.ds(k * L, L)]
                        sq = dlt * dlt
                        acc = sq if acc is None else acc + sq
                    res_v[res_row_base + e, pl.ds(p * L, L)] = acc
                acc = None
                for k in range(D // L):
                    dlt = xq[k] - xy_v[EDGES_PER_TILE + edge, pl.ds(k * L, L)]
                    sq = dlt * dlt
                    acc = sq if acc is None else acc + sq
                res_v[res_row_base + e, pl.ds(N_NEG * L, L)] = acc

        @pl.loop(0, CHUNKS_PER_TILE // 2)
        def _(t):
            c0 = 2 * t

            @pl.when(t > 0)
            def _():
                pltpu.make_async_copy(
                    res_v, res_hbm.at[pl.ds(edge_base, RES_ROWS)], sem_w
                ).wait()

            pltpu.make_async_copy(
                table_hbm.at[ni_v.at[pl.ds(c0 * CHUNK_NEG_ROWS,
                                           CHUNK_NEG_ROWS)]],
                neg0, sem_g).wait()
            compute_chunk(t, 0, neg0, 0)

            @pl.when(t < CHUNKS_PER_TILE // 2 - 1)
            def _():
                start_neg_gather(c0 + 2, neg0)

            pltpu.make_async_copy(
                table_hbm.at[ni_v.at[pl.ds((c0 + 1) * CHUNK_NEG_ROWS,
                                           CHUNK_NEG_ROWS)]],
                neg1, sem_g).wait()
            compute_chunk(t, 1, neg1, EDGES_PER_CHUNK)

            @pl.when(t < CHUNKS_PER_TILE // 2 - 1)
            def _():
                start_neg_gather(c0 + 3, neg1)

            pltpu.async_copy(
                res_v,
                res_hbm.at[pl.ds(edge_base + t * RES_ROWS, RES_ROWS)],
                sem_w)

        pltpu.make_async_copy(
            res_v, res_hbm.at[pl.ds(edge_base, RES_ROWS)], sem_w).wait()

    return sc_kernel(table, xs, ys, y_neg_flat)


def _tc_loss_body(r_ref, w_ref, o_ref):
    i = pl.program_id(0)
    blk = r_ref[...]

    row_ids = lax.broadcasted_iota(jnp.int32, (RES_LANES, RES_LANES // L), 0)
    col_ids = lax.broadcasted_iota(jnp.int32, (RES_LANES, RES_LANES // L), 1)
    sel = (row_ids // L == col_ids).astype(jnp.float32)

    dmat = jax.lax.dot_general(
        blk, sel, (((1,), (0,)), ((), ())),
        preferred_element_type=jnp.float32)

    dneg = dmat[:, :N_NEG]
    dpos = dmat[:, N_NEG:N_NEG + 1]

    p_pos = jnp.clip(1.0 / (1.0 + 0.25 * dpos), 1e-12, 1.0 - 1e-12)
    p_neg = jnp.clip(1.0 / (1.0 + 0.25 * dneg), 1e-12, 0.99)
    t_pos = jnp.log(p_pos) * 20.0
    t_neg = jnp.log(1.0 - p_neg)

    loss_b = 7.0 * jnp.sum(t_neg, axis=1) + t_pos[:, 0]
    part = jnp.sum(w_ref[0, :] * loss_b)

    @pl.when(i == 0)
    def _():
        o_ref[...] = jnp.zeros_like(o_ref)

    o_ref[...] += jnp.full((1, 1), -part, jnp.float32)


def _tc_loss(pair_partials, weights):
    grid = B // EDGE_BLK
    out = pl.pallas_call(
        _tc_loss_body,
        grid=(grid,),
        in_specs=[
            pl.BlockSpec((EDGE_BLK, RES_LANES), lambda i: (i, 0)),
            pl.BlockSpec((1, EDGE_BLK), lambda i: (0, i)),
        ],
        out_specs=pl.BlockSpec((1, 1), lambda i: (0, 0)),
        out_shape=jax.ShapeDtypeStruct((1, 1), jnp.float32),
    )(pair_partials, weights)
    return out[0, 0]


@jax.jit
def kernel(logits, xs, ys, y_neg, sample_edge_weight):
    partials = _sc_pair_partials(logits, xs, ys, y_neg.reshape(-1))
    return _tc_loss(partials, sample_edge_weight.reshape(1, B))
